# Initial kernel scaffold; baseline (speedup 1.0000x reference)
import jax
import jax.numpy as jnp
from jax.experimental import pallas as pl


def kernel(x, W1, b1, ln_g, ln_b, W2, b2, Wc1, bc1, Wc2, bc2):
    raise NotImplementedError("write your pallas kernel here")


# speedup vs baseline: 4.6653x; 4.6653x over previous
import functools

import jax
import jax.numpy as jnp
from jax import lax
from jax.experimental import pallas as pl
from jax.experimental.pallas import tpu as pltpu
from jax.experimental.pallas import tpu_sc as plsc

_B, _N, _CIN = 16, 8192, 5
_G, _K, _D = 128, 32, 128
_PAD = 16
_NC, _NS, _L = 2, 16, 16
_NW = _NC * _NS
_ROWS = _B * _G * _K
_RPW = _ROWS // _NW
_CHUNK = 128
_NCHUNK = _RPW // _CHUNK


def _fps_body(x0_ref, x1_ref, x2_ref, c0_ref, c1_ref, c2_ref, dist_ref):
    lane_n = lax.broadcasted_iota(jnp.int32, (_B, _N), 1)
    lane_g = lax.broadcasted_iota(jnp.int32, (_B, _G), 1)
    dist_ref[...] = jnp.full((_B, _N), 1e10, jnp.float32)

    def body(i, far):
        x0 = x0_ref[...]
        x1 = x1_ref[...]
        x2 = x2_ref[...]
        sel = lane_n == far
        cx = jnp.sum(jnp.where(sel, x0, 0.0), axis=1, keepdims=True)
        cy = jnp.sum(jnp.where(sel, x1, 0.0), axis=1, keepdims=True)
        cz = jnp.sum(jnp.where(sel, x2, 0.0), axis=1, keepdims=True)
        hit = lane_g == i
        c0_ref[...] = jnp.where(hit, cx, c0_ref[...])
        c1_ref[...] = jnp.where(hit, cy, c1_ref[...])
        c2_ref[...] = jnp.where(hit, cz, c2_ref[...])
        dx = x0 - cx
        dy = x1 - cy
        dz = x2 - cz
        d = dx * dx + dy * dy + dz * dz
        dd = dist_ref[...]
        dd = jnp.where(d < dd, d, dd)
        dist_ref[...] = dd
        m = jnp.max(dd, axis=1, keepdims=True)
        far_new = jnp.min(jnp.where(dd == m, lane_n, _N), axis=1, keepdims=True)
        return far_new.astype(jnp.int32)

    lax.fori_loop(0, _G, body, jnp.zeros((_B, 1), jnp.int32))


_fps = pl.pallas_call(
    _fps_body,
    out_shape=[jax.ShapeDtypeStruct((_B, _G), jnp.float32)] * 3,
    scratch_shapes=[pltpu.VMEM((_B, _N), jnp.float32)],
)


def _knn_body(x_ref, c0_ref, c1_ref, c2_ref, gout_ref, d_ref):
    xyz = x_ref[0]
    c0 = c0_ref[0]
    c1 = c1_ref[0]
    c2 = c2_ref[0]
    ct = jnp.concatenate([c0, c1, c2], axis=0)
    mm = jnp.dot(xyz, ct, preferred_element_type=jnp.float32)
    cs = c0 * c0 + c1 * c1 + c2 * c2
    x0 = xyz[:, 0:1]
    x1 = xyz[:, 1:2]
    x2 = xyz[:, 2:3]
    ps = x0 * x0 + x1 * x1 + x2 * x2
    d = -2.0 * mm
    d = d + cs
    d = d + ps
    d_ref[...] = d
    rows = lax.broadcasted_iota(jnp.int32, (_N, _G), 0)

    def body(k, _):
        dd = d_ref[...]
        m = jnp.min(dd, axis=0, keepdims=True)
        ii = jnp.min(jnp.where(dd == m, rows, _N), axis=0, keepdims=True)
        gout_ref[0, pl.ds(k, 1), :] = ii
        d_ref[...] = jnp.where(rows == ii, jnp.inf, dd)
        return 0

    lax.fori_loop(0, _K, body, 0)


_knn = pl.pallas_call(
    _knn_body,
    grid=(_B,),
    in_specs=[
        pl.BlockSpec((1, _N, 3), lambda b: (b, 0, 0)),
        pl.BlockSpec((1, 1, _G), lambda b: (b, 0, 0)),
        pl.BlockSpec((1, 1, _G), lambda b: (b, 0, 0)),
        pl.BlockSpec((1, 1, _G), lambda b: (b, 0, 0)),
    ],
    out_specs=pl.BlockSpec((1, _K, _G), lambda b: (b, 0, 0)),
    out_shape=jax.ShapeDtypeStruct((_B, _K, _G), jnp.int32),
    scratch_shapes=[pltpu.VMEM((_N, _G), jnp.float32)],
    compiler_params=pltpu.CompilerParams(
        dimension_semantics=("arbitrary",)),
)


def _sc_gather_body(tbl_ref, idx_ref, out_ref, idx_v, rows_v, sem):
    wid = lax.axis_index("s") * _NC + lax.axis_index("c")
    pltpu.sync_copy(idx_ref.at[pl.ds(wid * _NCHUNK, _NCHUNK)], idx_v)
    copies = [
        pltpu.async_copy(
            tbl_ref.at[idx_v.at[j]],
            rows_v.at[pl.ds(j * _CHUNK, _CHUNK)],
            sem,
        )
        for j in range(_NCHUNK)
    ]
    for cp in copies:
        cp.wait()
    pltpu.sync_copy(rows_v, out_ref.at[pl.ds(wid * _RPW, _RPW)])


_sc_gather_cache = []


def _sc_gather(tbl, flat_idx):
    if not _sc_gather_cache:
        _sc_gather_cache.append(functools.partial(
            pl.kernel,
            out_type=jax.ShapeDtypeStruct((_ROWS, _PAD), jnp.float32),
            mesh=plsc.VectorSubcoreMesh(
                core_axis_name="c", subcore_axis_name="s"),
            scratch_types=[
                pltpu.VMEM((_NCHUNK, _CHUNK), jnp.int32),
                pltpu.VMEM((_RPW, _PAD), jnp.float32),
                pltpu.SemaphoreType.DMA,
            ],
            compiler_params=pltpu.CompilerParams(use_tc_tiling_on_sc=False),
        )(_sc_gather_body))
    return _sc_gather_cache[0](tbl, flat_idx)


def _mlp_body(pp_ref, cp_ref, w1_ref, b1_ref, g_ref, be_ref, w2_ref, b2_ref,
              wc1_ref, bc1_ref, wc2_ref, bc2_ref, out_ref):
    pp = pp_ref[...]
    cp = cp_ref[...]
    h = jnp.dot(pp, w1_ref[...], preferred_element_type=jnp.float32)
    hc = jnp.dot(cp, w1_ref[...], preferred_element_type=jnp.float32)
    h = h.reshape(_TOK, _K, _D) - hc[:, None, :]
    h = h + b1_ref[...]
    mu = jnp.mean(h, axis=2, keepdims=True)
    var = jnp.mean((h - mu) * (h - mu), axis=2, keepdims=True)
    h = (h - mu) / jnp.sqrt(var + 1e-5) * g_ref[...] + be_ref[...]
    h = jnp.maximum(h, 0.0)
    h2 = jnp.dot(h.reshape(_TOK * _K, _D), w2_ref[...],
                 preferred_element_type=jnp.float32) + b2_ref[...]
    t = jnp.max(h2.reshape(_TOK, _K, _D), axis=1)
    pe = jnp.maximum(
        jnp.dot(cp, wc1_ref[...], preferred_element_type=jnp.float32)
        + bc1_ref[...], 0.0)
    pe = jnp.dot(pe, wc2_ref[...], preferred_element_type=jnp.float32) \
        + bc2_ref[...]
    out_ref[...] = t + pe


_NMLP = 8
_TOK = _B * _G // _NMLP

_mlp = pl.pallas_call(
    _mlp_body,
    grid=(_NMLP,),
    in_specs=[
        pl.BlockSpec((_TOK * _K, _PAD), lambda i: (i, 0)),
        pl.BlockSpec((_TOK, _PAD), lambda i: (i, 0)),
        pl.BlockSpec((_PAD, _D), lambda i: (0, 0)),
        pl.BlockSpec((1, _D), lambda i: (0, 0)),
        pl.BlockSpec((1, _D), lambda i: (0, 0)),
        pl.BlockSpec((1, _D), lambda i: (0, 0)),
        pl.BlockSpec((_D, _D), lambda i: (0, 0)),
        pl.BlockSpec((1, _D), lambda i: (0, 0)),
        pl.BlockSpec((_PAD, _D), lambda i: (0, 0)),
        pl.BlockSpec((1, _D), lambda i: (0, 0)),
        pl.BlockSpec((_D, _D), lambda i: (0, 0)),
        pl.BlockSpec((1, _D), lambda i: (0, 0)),
    ],
    out_specs=pl.BlockSpec((_TOK, _D), lambda i: (i, 0)),
    out_shape=jax.ShapeDtypeStruct((_B * _G, _D), jnp.float32),
    compiler_params=pltpu.CompilerParams(
        dimension_semantics=("arbitrary",)),
)


def kernel(x, W1, b1, ln_g, ln_b, W2, b2, Wc1, bc1, Wc2, bc2):
    xyz = x[:, :, :3]
    xt = jnp.transpose(xyz, (2, 0, 1))
    c0, c1, c2 = _fps(xt[0], xt[1], xt[2])
    gidx_t = _knn(xyz, c0[:, None, :], c1[:, None, :], c2[:, None, :])
    group_idx = jnp.transpose(gidx_t, (0, 2, 1))
    centers = jnp.stack([c0, c1, c2], axis=-1)

    offs = (jnp.arange(_B, dtype=jnp.int32) * _N)[:, None, None]
    flat_idx = (group_idx + offs).reshape(_ROWS // _CHUNK, _CHUNK)
    tbl = jnp.pad(x.reshape(_B * _N, _CIN), ((0, 0), (0, _PAD - _CIN)))
    rows = _sc_gather(tbl, flat_idx)
    patches = rows[:, :_CIN].reshape(_B, _G, _K, _CIN)

    cpad = jnp.pad(centers.reshape(_B * _G, 3), ((0, 0), (0, _PAD - 3)))
    w1p = jnp.pad(W1, ((0, _PAD - _CIN), (0, 0)))
    wc1p = jnp.pad(Wc1, ((0, _PAD - 3), (0, 0)))
    tokens2 = _mlp(rows, cpad, w1p, b1.reshape(1, _D), ln_g.reshape(1, _D),
                   ln_b.reshape(1, _D), W2, b2.reshape(1, _D), wc1p,
                   bc1.reshape(1, _D), Wc2, bc2.reshape(1, _D))
    tokens = tokens2.reshape(_B, _G, _D)
    return (tokens, centers, patches, group_idx)


# speedup vs baseline: 6.7094x; 1.4381x over previous
import functools

import jax
import jax.numpy as jnp
from jax import lax
from jax.experimental import pallas as pl
from jax.experimental.pallas import tpu as pltpu
from jax.experimental.pallas import tpu_sc as plsc

_B, _N, _CIN = 16, 8192, 5
_G, _K, _D = 128, 32, 128
_PAD = 16
_NC, _NS, _L = 2, 16, 16
_NW = _NC * _NS
_ROWS = _B * _G * _K
_RPW = _ROWS // _NW
_CHUNK = 128
_NCHUNK = _RPW // _CHUNK


def _fps_body(x0_ref, x1_ref, x2_ref, c0_ref, c1_ref, c2_ref, dist_ref):
    lane_n = lax.broadcasted_iota(jnp.int32, (_B, _N), 1)
    lane_g = lax.broadcasted_iota(jnp.int32, (_B, _G), 1)
    dist_ref[...] = jnp.full((_B, _N), 1e10, jnp.float32)

    def body(i, far):
        x0 = x0_ref[...]
        x1 = x1_ref[...]
        x2 = x2_ref[...]
        sel = lane_n == far
        cx = jnp.sum(jnp.where(sel, x0, 0.0), axis=1, keepdims=True)
        cy = jnp.sum(jnp.where(sel, x1, 0.0), axis=1, keepdims=True)
        cz = jnp.sum(jnp.where(sel, x2, 0.0), axis=1, keepdims=True)
        hit = lane_g == i
        c0_ref[...] = jnp.where(hit, cx, c0_ref[...])
        c1_ref[...] = jnp.where(hit, cy, c1_ref[...])
        c2_ref[...] = jnp.where(hit, cz, c2_ref[...])
        dx = x0 - cx
        dy = x1 - cy
        dz = x2 - cz
        d = dx * dx + dy * dy + dz * dz
        dd = dist_ref[...]
        dd = jnp.where(d < dd, d, dd)
        dist_ref[...] = dd
        m = jnp.max(dd, axis=1, keepdims=True)
        far_new = jnp.min(jnp.where(dd == m, lane_n, _N), axis=1, keepdims=True)
        return far_new.astype(jnp.int32)

    lax.fori_loop(0, _G, body, jnp.zeros((_B, 1), jnp.int32))


_fps = pl.pallas_call(
    _fps_body,
    out_shape=[jax.ShapeDtypeStruct((_B, _G), jnp.float32)] * 3,
    scratch_shapes=[pltpu.VMEM((_B, _N), jnp.float32)],
)


def _knn_body(x_ref, c0_ref, c1_ref, c2_ref, gout_ref, d_ref):
    xyz = x_ref[0]
    c0 = c0_ref[0]
    c1 = c1_ref[0]
    c2 = c2_ref[0]
    ct = jnp.concatenate([c0, c1, c2], axis=0)
    mm = jnp.dot(xyz, ct, preferred_element_type=jnp.float32)
    cs = c0 * c0 + c1 * c1 + c2 * c2
    x0 = xyz[:, 0:1]
    x1 = xyz[:, 1:2]
    x2 = xyz[:, 2:3]
    ps = x0 * x0 + x1 * x1 + x2 * x2
    d = -2.0 * mm
    d = d + cs
    d = d + ps
    d_ref[...] = d
    rows = lax.broadcasted_iota(jnp.int32, (_N, _G), 0)

    def _hmin(a):
        return jnp.min(jnp.min(a.reshape(64, _N // 64, _G), axis=0),
                       axis=0, keepdims=True)

    def _emit(k, dd, m):
        ii = jnp.min(jnp.min(jnp.where(dd == m, rows, _N)
                             .reshape(64, _N // 64, _G), axis=0),
                     axis=0, keepdims=True)
        gout_ref[0, pl.ds(k, 1), :] = ii
        return ii

    def body(k, m):
        dd = d_ref[...]
        ii = _emit(k, dd, m)
        dd2 = jnp.where(rows == ii, jnp.inf, dd)
        d_ref[...] = dd2
        return _hmin(dd2)

    m_last = lax.fori_loop(0, _K - 1, body, _hmin(d))
    _emit(_K - 1, d_ref[...], m_last)


_knn = pl.pallas_call(
    _knn_body,
    grid=(_B,),
    in_specs=[
        pl.BlockSpec((1, _N, 3), lambda b: (b, 0, 0)),
        pl.BlockSpec((1, 1, _G), lambda b: (b, 0, 0)),
        pl.BlockSpec((1, 1, _G), lambda b: (b, 0, 0)),
        pl.BlockSpec((1, 1, _G), lambda b: (b, 0, 0)),
    ],
    out_specs=pl.BlockSpec((1, _K, _G), lambda b: (b, 0, 0)),
    out_shape=jax.ShapeDtypeStruct((_B, _K, _G), jnp.int32),
    scratch_shapes=[pltpu.VMEM((_N, _G), jnp.float32)],
    compiler_params=pltpu.CompilerParams(
        dimension_semantics=("arbitrary",)),
)


def _sc_gather_body(tbl_ref, idx_ref, out_ref, idx_v, rows_v, sem):
    wid = lax.axis_index("s") * _NC + lax.axis_index("c")
    pltpu.sync_copy(idx_ref.at[pl.ds(wid * _NCHUNK, _NCHUNK)], idx_v)
    copies = [
        pltpu.async_copy(
            tbl_ref.at[idx_v.at[j]],
            rows_v.at[pl.ds(j * _CHUNK, _CHUNK)],
            sem,
        )
        for j in range(_NCHUNK)
    ]
    for cp in copies:
        cp.wait()
    pltpu.sync_copy(rows_v, out_ref.at[pl.ds(wid * _RPW, _RPW)])


_sc_gather_cache = []


def _sc_gather(tbl, flat_idx):
    if not _sc_gather_cache:
        _sc_gather_cache.append(functools.partial(
            pl.kernel,
            out_type=jax.ShapeDtypeStruct((_ROWS, _PAD), jnp.float32),
            mesh=plsc.VectorSubcoreMesh(
                core_axis_name="c", subcore_axis_name="s"),
            scratch_types=[
                pltpu.VMEM((_NCHUNK, _CHUNK), jnp.int32),
                pltpu.VMEM((_RPW, _PAD), jnp.float32),
                pltpu.SemaphoreType.DMA,
            ],
            compiler_params=pltpu.CompilerParams(use_tc_tiling_on_sc=False),
        )(_sc_gather_body))
    return _sc_gather_cache[0](tbl, flat_idx)


def _mlp_body(pp_ref, cp_ref, w1_ref, b1_ref, g_ref, be_ref, w2_ref, b2_ref,
              wc1_ref, bc1_ref, wc2_ref, bc2_ref, out_ref):
    pp = pp_ref[...]
    cp = cp_ref[...]
    h = jnp.dot(pp, w1_ref[...], preferred_element_type=jnp.float32)
    hc = jnp.dot(cp, w1_ref[...], preferred_element_type=jnp.float32)
    h = h.reshape(_TOK, _K, _D) - hc[:, None, :]
    h = h + b1_ref[...]
    mu = jnp.mean(h, axis=2, keepdims=True)
    var = jnp.mean((h - mu) * (h - mu), axis=2, keepdims=True)
    h = (h - mu) / jnp.sqrt(var + 1e-5) * g_ref[...] + be_ref[...]
    h = jnp.maximum(h, 0.0)
    h2 = jnp.dot(h.reshape(_TOK * _K, _D), w2_ref[...],
                 preferred_element_type=jnp.float32) + b2_ref[...]
    t = jnp.max(h2.reshape(_TOK, _K, _D), axis=1)
    pe = jnp.maximum(
        jnp.dot(cp, wc1_ref[...], preferred_element_type=jnp.float32)
        + bc1_ref[...], 0.0)
    pe = jnp.dot(pe, wc2_ref[...], preferred_element_type=jnp.float32) \
        + bc2_ref[...]
    out_ref[...] = t + pe


_NMLP = 8
_TOK = _B * _G // _NMLP

_mlp = pl.pallas_call(
    _mlp_body,
    grid=(_NMLP,),
    in_specs=[
        pl.BlockSpec((_TOK * _K, _PAD), lambda i: (i, 0)),
        pl.BlockSpec((_TOK, _PAD), lambda i: (i, 0)),
        pl.BlockSpec((_PAD, _D), lambda i: (0, 0)),
        pl.BlockSpec((1, _D), lambda i: (0, 0)),
        pl.BlockSpec((1, _D), lambda i: (0, 0)),
        pl.BlockSpec((1, _D), lambda i: (0, 0)),
        pl.BlockSpec((_D, _D), lambda i: (0, 0)),
        pl.BlockSpec((1, _D), lambda i: (0, 0)),
        pl.BlockSpec((_PAD, _D), lambda i: (0, 0)),
        pl.BlockSpec((1, _D), lambda i: (0, 0)),
        pl.BlockSpec((_D, _D), lambda i: (0, 0)),
        pl.BlockSpec((1, _D), lambda i: (0, 0)),
    ],
    out_specs=pl.BlockSpec((_TOK, _D), lambda i: (i, 0)),
    out_shape=jax.ShapeDtypeStruct((_B * _G, _D), jnp.float32),
    compiler_params=pltpu.CompilerParams(
        dimension_semantics=("arbitrary",)),
)


def kernel(x, W1, b1, ln_g, ln_b, W2, b2, Wc1, bc1, Wc2, bc2):
    xyz = x[:, :, :3]
    xt = jnp.transpose(xyz, (2, 0, 1))
    c0, c1, c2 = _fps(xt[0], xt[1], xt[2])
    gidx_t = _knn(xyz, c0[:, None, :], c1[:, None, :], c2[:, None, :])
    group_idx = jnp.transpose(gidx_t, (0, 2, 1))
    centers = jnp.stack([c0, c1, c2], axis=-1)

    offs = (jnp.arange(_B, dtype=jnp.int32) * _N)[:, None, None]
    flat_idx = (group_idx + offs).reshape(_ROWS // _CHUNK, _CHUNK)
    tbl = jnp.pad(x.reshape(_B * _N, _CIN), ((0, 0), (0, _PAD - _CIN)))
    rows = _sc_gather(tbl, flat_idx)
    patches = rows[:, :_CIN].reshape(_B, _G, _K, _CIN)

    cpad = jnp.pad(centers.reshape(_B * _G, 3), ((0, 0), (0, _PAD - 3)))
    w1p = jnp.pad(W1, ((0, _PAD - _CIN), (0, 0)))
    wc1p = jnp.pad(Wc1, ((0, _PAD - 3), (0, 0)))
    tokens2 = _mlp(rows, cpad, w1p, b1.reshape(1, _D), ln_g.reshape(1, _D),
                   ln_b.reshape(1, _D), W2, b2.reshape(1, _D), wc1p,
                   bc1.reshape(1, _D), Wc2, bc2.reshape(1, _D))
    tokens = tokens2.reshape(_B, _G, _D)
    return (tokens, centers, patches, group_idx)


# speedup vs baseline: 7.5529x; 1.1257x over previous
import functools

import jax
import jax.numpy as jnp
from jax import lax
from jax.experimental import pallas as pl
from jax.experimental.pallas import tpu as pltpu
from jax.experimental.pallas import tpu_sc as plsc

_B, _N, _CIN = 16, 8192, 5
_G, _K, _D = 128, 32, 128
_PAD = 16
_NC, _NS, _L = 2, 16, 16
_NW = _NC * _NS
_ROWS = _B * _G * _K
_RPW = _ROWS // _NW
_CHUNK = 128
_NCHUNK = _RPW // _CHUNK


def _fps_body(x0_ref, x1_ref, x2_ref, c0_ref, c1_ref, c2_ref, dist_ref):
    lane_n = lax.broadcasted_iota(jnp.int32, (_B, _N), 1)
    lane_g = lax.broadcasted_iota(jnp.int32, (_B, _G), 1)
    dist_ref[...] = jnp.full((_B, _N), 1e10, jnp.float32)

    def body(i, far):
        x0 = x0_ref[...]
        x1 = x1_ref[...]
        x2 = x2_ref[...]
        sel = lane_n == far
        cx = jnp.sum(jnp.where(sel, x0, 0.0), axis=1, keepdims=True)
        cy = jnp.sum(jnp.where(sel, x1, 0.0), axis=1, keepdims=True)
        cz = jnp.sum(jnp.where(sel, x2, 0.0), axis=1, keepdims=True)
        hit = lane_g == i
        c0_ref[...] = jnp.where(hit, cx, c0_ref[...])
        c1_ref[...] = jnp.where(hit, cy, c1_ref[...])
        c2_ref[...] = jnp.where(hit, cz, c2_ref[...])
        dx = x0 - cx
        dy = x1 - cy
        dz = x2 - cz
        d = dx * dx + dy * dy + dz * dz
        dd = dist_ref[...]
        dd = jnp.where(d < dd, d, dd)
        dist_ref[...] = dd
        m = jnp.max(dd, axis=1, keepdims=True)
        far_new = jnp.min(jnp.where(dd == m, lane_n, _N), axis=1, keepdims=True)
        return far_new.astype(jnp.int32)

    lax.fori_loop(0, _G, body, jnp.zeros((_B, 1), jnp.int32))


_fps = pl.pallas_call(
    _fps_body,
    out_shape=[jax.ShapeDtypeStruct((_B, _G), jnp.float32)] * 3,
    scratch_shapes=[pltpu.VMEM((_B, _N), jnp.float32)],
)


def _knn_body(x0_ref, x1_ref, x2_ref, c0_ref, c1_ref, c2_ref, gout_ref,
              d_ref):
    x0 = x0_ref[0]
    x1 = x1_ref[0]
    x2 = x2_ref[0]
    c0 = c0_ref[0]
    c1 = c1_ref[0]
    c2 = c2_ref[0]
    cs = c0 * c0 + c1 * c1 + c2 * c2
    xyz = jnp.transpose(jnp.concatenate([x0, x1, x2], axis=0), (1, 0))
    a0 = xyz[:, 0:1]
    a1 = xyz[:, 1:2]
    a2 = xyz[:, 2:3]
    ps = a0 * a0 + a1 * a1 + a2 * a2
    ct = jnp.concatenate([c0, c1, c2], axis=0)
    mm = jnp.dot(xyz, ct, preferred_element_type=jnp.float32)
    d = -2.0 * mm
    d = d + cs
    d = d + ps
    d_ref[...] = d
    rows = lax.broadcasted_iota(jnp.int32, (_N, _G), 0)

    def _hmin(a):
        return jnp.min(jnp.min(a.reshape(64, _N // 64, _G), axis=0),
                       axis=0, keepdims=True)

    def _emit(k, dd, m):
        ii = jnp.min(jnp.min(jnp.where(dd == m, rows, _N)
                             .reshape(64, _N // 64, _G), axis=0),
                     axis=0, keepdims=True)
        gout_ref[0, pl.ds(k, 1), :] = ii
        return ii

    def body(k, m):
        dd = d_ref[...]
        ii = _emit(k, dd, m)
        dd2 = jnp.where(rows == ii, jnp.inf, dd)
        d_ref[...] = dd2
        return _hmin(dd2)

    m_last = lax.fori_loop(0, _K - 1, body, _hmin(d))
    _emit(_K - 1, d_ref[...], m_last)


_knn = pl.pallas_call(
    _knn_body,
    grid=(_B,),
    in_specs=[
        pl.BlockSpec((1, 1, _N), lambda b: (b, 0, 0)),
        pl.BlockSpec((1, 1, _N), lambda b: (b, 0, 0)),
        pl.BlockSpec((1, 1, _N), lambda b: (b, 0, 0)),
        pl.BlockSpec((1, 1, _G), lambda b: (b, 0, 0)),
        pl.BlockSpec((1, 1, _G), lambda b: (b, 0, 0)),
        pl.BlockSpec((1, 1, _G), lambda b: (b, 0, 0)),
    ],
    out_specs=pl.BlockSpec((1, _K, _G), lambda b: (b, 0, 0)),
    out_shape=jax.ShapeDtypeStruct((_B, _K, _G), jnp.int32),
    scratch_shapes=[pltpu.VMEM((_N, _G), jnp.float32)],
    compiler_params=pltpu.CompilerParams(
        dimension_semantics=("arbitrary",)),
)


def _sc_gather_body(tbl_ref, idx_ref, out_ref, idx_v, rows_v, sem):
    wid = lax.axis_index("s") * _NC + lax.axis_index("c")
    pltpu.sync_copy(idx_ref.at[pl.ds(wid * _NCHUNK, _NCHUNK)], idx_v)
    copies = [
        pltpu.async_copy(
            tbl_ref.at[idx_v.at[j]],
            rows_v.at[pl.ds(j * _CHUNK, _CHUNK)],
            sem,
        )
        for j in range(_NCHUNK)
    ]
    for cp in copies:
        cp.wait()
    pltpu.sync_copy(rows_v, out_ref.at[pl.ds(wid * _RPW, _RPW)])


_sc_gather_cache = []


def _sc_gather(tbl, flat_idx):
    if not _sc_gather_cache:
        _sc_gather_cache.append(functools.partial(
            pl.kernel,
            out_type=jax.ShapeDtypeStruct((_ROWS, _PAD), jnp.float32),
            mesh=plsc.VectorSubcoreMesh(
                core_axis_name="c", subcore_axis_name="s"),
            scratch_types=[
                pltpu.VMEM((_NCHUNK, _CHUNK), jnp.int32),
                pltpu.VMEM((_RPW, _PAD), jnp.float32),
                pltpu.SemaphoreType.DMA,
            ],
            compiler_params=pltpu.CompilerParams(use_tc_tiling_on_sc=False),
        )(_sc_gather_body))
    return _sc_gather_cache[0](tbl, flat_idx)


def _mlp_body(pp_ref, cp_ref, w1_ref, b1_ref, g_ref, be_ref, w2_ref, b2_ref,
              wc1_ref, bc1_ref, wc2_ref, bc2_ref, out_ref):
    pp = pp_ref[...]
    cp = cp_ref[...]
    h = jnp.dot(pp, w1_ref[...], preferred_element_type=jnp.float32)
    hc = jnp.dot(cp, w1_ref[...], preferred_element_type=jnp.float32)
    h = h.reshape(_TOK, _K, _D) - hc[:, None, :]
    h = h + b1_ref[...]
    mu = jnp.mean(h, axis=2, keepdims=True)
    var = jnp.mean((h - mu) * (h - mu), axis=2, keepdims=True)
    h = (h - mu) / jnp.sqrt(var + 1e-5) * g_ref[...] + be_ref[...]
    h = jnp.maximum(h, 0.0)
    h2 = jnp.dot(h.reshape(_TOK * _K, _D), w2_ref[...],
                 preferred_element_type=jnp.float32) + b2_ref[...]
    t = jnp.max(h2.reshape(_TOK, _K, _D), axis=1)
    pe = jnp.maximum(
        jnp.dot(cp, wc1_ref[...], preferred_element_type=jnp.float32)
        + bc1_ref[...], 0.0)
    pe = jnp.dot(pe, wc2_ref[...], preferred_element_type=jnp.float32) \
        + bc2_ref[...]
    out_ref[...] = t + pe


_NMLP = 8
_TOK = _B * _G // _NMLP

_mlp = pl.pallas_call(
    _mlp_body,
    grid=(_NMLP,),
    in_specs=[
        pl.BlockSpec((_TOK * _K, _PAD), lambda i: (i, 0)),
        pl.BlockSpec((_TOK, _PAD), lambda i: (i, 0)),
        pl.BlockSpec((_PAD, _D), lambda i: (0, 0)),
        pl.BlockSpec((1, _D), lambda i: (0, 0)),
        pl.BlockSpec((1, _D), lambda i: (0, 0)),
        pl.BlockSpec((1, _D), lambda i: (0, 0)),
        pl.BlockSpec((_D, _D), lambda i: (0, 0)),
        pl.BlockSpec((1, _D), lambda i: (0, 0)),
        pl.BlockSpec((_PAD, _D), lambda i: (0, 0)),
        pl.BlockSpec((1, _D), lambda i: (0, 0)),
        pl.BlockSpec((_D, _D), lambda i: (0, 0)),
        pl.BlockSpec((1, _D), lambda i: (0, 0)),
    ],
    out_specs=pl.BlockSpec((_TOK, _D), lambda i: (i, 0)),
    out_shape=jax.ShapeDtypeStruct((_B * _G, _D), jnp.float32),
    compiler_params=pltpu.CompilerParams(
        dimension_semantics=("arbitrary",)),
)


def kernel(x, W1, b1, ln_g, ln_b, W2, b2, Wc1, bc1, Wc2, bc2):
    xyz = x[:, :, :3]
    xt = jnp.transpose(xyz, (2, 0, 1))
    c0, c1, c2 = _fps(xt[0], xt[1], xt[2])
    gidx_t = _knn(xt[0][:, None, :], xt[1][:, None, :], xt[2][:, None, :],
                  c0[:, None, :], c1[:, None, :], c2[:, None, :])
    group_idx = jnp.transpose(gidx_t, (0, 2, 1))
    centers = jnp.stack([c0, c1, c2], axis=-1)

    offs = (jnp.arange(_B, dtype=jnp.int32) * _N)[:, None, None]
    flat_idx = (group_idx + offs).reshape(_ROWS // _CHUNK, _CHUNK)
    tbl = jnp.pad(x.reshape(_B * _N, _CIN), ((0, 0), (0, _PAD - _CIN)))
    rows = _sc_gather(tbl, flat_idx)
    patches = rows[:, :_CIN].reshape(_B, _G, _K, _CIN)

    cpad = jnp.pad(centers.reshape(_B * _G, 3), ((0, 0), (0, _PAD - 3)))
    w1p = jnp.pad(W1, ((0, _PAD - _CIN), (0, 0)))
    wc1p = jnp.pad(Wc1, ((0, _PAD - 3), (0, 0)))
    tokens2 = _mlp(rows, cpad, w1p, b1.reshape(1, _D), ln_g.reshape(1, _D),
                   ln_b.reshape(1, _D), W2, b2.reshape(1, _D), wc1p,
                   bc1.reshape(1, _D), Wc2, bc2.reshape(1, _D))
    tokens = tokens2.reshape(_B, _G, _D)
    return (tokens, centers, patches, group_idx)
